# baseline (device time: 43346 ns/iter reference)
import jax
import jax.numpy as jnp
from jax import lax
from jax.experimental import pallas as pl
from jax.experimental.pallas import tpu as pltpu

N_DEV = 4


def kernel(x):
    m_per, n = x.shape
    m_half = m_per // 2

    def body(x_ref, out_ref, send_sems, recv_sems, local_sem):
        my_pos = lax.axis_index("i")
        left = (my_pos - 1) % N_DEV
        right = (my_pos + 1) % N_DEV

        barrier_sem = pltpu.get_barrier_semaphore()
        for nbr in [left, right]:
            pl.semaphore_signal(
                barrier_sem, inc=1,
                device_id=(nbr,), device_id_type=pl.DeviceIdType.MESH,
            )
        pl.semaphore_wait(barrier_sem, 2)

        x_top = x_ref.at[pl.ds(0, m_half), :]
        x_bot = x_ref.at[pl.ds(m_half, m_half), :]
        my_top = out_ref.at[pl.ds(my_pos * m_per, m_half), :]
        my_bot = out_ref.at[pl.ds(my_pos * m_per + m_half, m_half), :]

        def remote(src, dst, s, r, tgt):
            return pltpu.make_async_remote_copy(
                src_ref=src, dst_ref=dst,
                send_sem=send_sems.at[s], recv_sem=recv_sems.at[r],
                device_id=(tgt,), device_id_type=pl.DeviceIdType.MESH,
            )

        a1 = remote(x_top, my_top, 0, 0, right)
        b1 = remote(x_bot, my_bot, 3, 3, left)
        a2 = remote(x_bot, my_bot, 1, 1, right)
        b2 = remote(x_top, my_top, 4, 4, left)
        a1.start()
        b1.start()
        a2.start()
        b2.start()

        local_copy = pltpu.make_async_copy(
            x_ref, out_ref.at[pl.ds(my_pos * m_per, m_per), :], local_sem
        )
        local_copy.start()

        a1.wait_recv()
        diag_top = out_ref.at[pl.ds(left * m_per, m_half), :]
        a3 = remote(diag_top, diag_top, 2, 2, right)
        a3.start()

        b1.wait_recv()
        diag_bot = out_ref.at[pl.ds(right * m_per + m_half, m_half), :]
        b3 = remote(diag_bot, diag_bot, 5, 5, left)
        b3.start()

        local_copy.wait()
        a1.wait_send()
        b1.wait_send()
        a2.wait()
        b2.wait()
        a3.wait()
        b3.wait()

    return pl.pallas_call(
        body,
        out_shape=jax.ShapeDtypeStruct((N_DEV * m_per, n), x.dtype),
        in_specs=[pl.BlockSpec(memory_space=pltpu.VMEM)],
        out_specs=pl.BlockSpec(memory_space=pltpu.MemorySpace.HBM),
        scratch_shapes=[
            pltpu.SemaphoreType.DMA((6,)),
            pltpu.SemaphoreType.DMA((6,)),
            pltpu.SemaphoreType.DMA,
        ],
        compiler_params=pltpu.CompilerParams(
            collective_id=0,
            skip_device_barrier=True,
            disable_bounds_checks=True,
        ),
    )(x)


# device time: 43324 ns/iter; 1.0005x vs baseline; 1.0005x over previous
import jax
import jax.numpy as jnp
from jax import lax
from jax.experimental import pallas as pl
from jax.experimental.pallas import tpu as pltpu

N_DEV = 4


def kernel(x):
    m_per, n = x.shape
    m_half = m_per // 2

    def body(x_ref, out_ref, send_sems, recv_sems, local_sem):
        my_pos = lax.axis_index("i")
        left = (my_pos - 1) % N_DEV
        right = (my_pos + 1) % N_DEV

        barrier_sem = pltpu.get_barrier_semaphore()
        for nbr in [left, right]:
            pl.semaphore_signal(
                barrier_sem, inc=1,
                device_id=(nbr,), device_id_type=pl.DeviceIdType.MESH,
            )
        pl.semaphore_wait(barrier_sem, 2)

        x_top = x_ref.at[pl.ds(0, m_half), :]
        x_bot = x_ref.at[pl.ds(m_half, m_half), :]
        my_top = out_ref.at[pl.ds(my_pos * m_per, m_half), :]
        my_bot = out_ref.at[pl.ds(my_pos * m_per + m_half, m_half), :]

        def remote(src, dst, s, r, tgt):
            return pltpu.make_async_remote_copy(
                src_ref=src, dst_ref=dst,
                send_sem=send_sems.at[s], recv_sem=recv_sems.at[r],
                device_id=(tgt,), device_id_type=pl.DeviceIdType.MESH,
            )

        a1 = remote(x_top, my_top, 0, 0, right)
        b1 = remote(x_bot, my_bot, 3, 3, left)
        a2 = remote(x_bot, my_bot, 1, 1, right)
        b2 = remote(x_top, my_top, 4, 4, left)
        a1.start()
        b1.start()
        a2.start()
        b2.start()

        local_copy = pltpu.make_async_copy(
            x_ref, out_ref.at[pl.ds(my_pos * m_per, m_per), :], local_sem
        )
        local_copy.start()

        a1.wait_recv()
        diag_top = out_ref.at[pl.ds(left * m_per, m_half), :]
        a3 = remote(diag_top, diag_top, 2, 2, right)
        a3.start()

        b1.wait_recv()
        diag_bot = out_ref.at[pl.ds(right * m_per + m_half, m_half), :]
        b3 = remote(diag_bot, diag_bot, 5, 5, left)
        b3.start()

        local_copy.wait()
        a1.wait_send()
        b1.wait_send()
        a2.wait()
        b2.wait()
        a3.wait()
        b3.wait()

    return pl.pallas_call(
        body,
        out_shape=jax.ShapeDtypeStruct((N_DEV * m_per, n), x.dtype),
        in_specs=[pl.BlockSpec(memory_space=pltpu.MemorySpace.HBM)],
        out_specs=pl.BlockSpec(memory_space=pltpu.MemorySpace.HBM),
        scratch_shapes=[
            pltpu.SemaphoreType.DMA((6,)),
            pltpu.SemaphoreType.DMA((6,)),
            pltpu.SemaphoreType.DMA,
        ],
        compiler_params=pltpu.CompilerParams(
            collective_id=0,
            skip_device_barrier=True,
            disable_bounds_checks=True,
        ),
    )(x)


# device time: 42219 ns/iter; 1.0267x vs baseline; 1.0262x over previous
import jax
import jax.numpy as jnp
from jax import lax
from jax.experimental import pallas as pl
from jax.experimental.pallas import tpu as pltpu

N_DEV = 4


def kernel(x):
    m_per, n = x.shape
    m_half = m_per // 2

    def body(x_ref, out_ref, send_sems, recv_sems, local_sem):
        my_pos = lax.axis_index("i")
        left = (my_pos - 1) % N_DEV
        right = (my_pos + 1) % N_DEV

        barrier_sem = pltpu.get_barrier_semaphore()
        for nbr in [left, right]:
            pl.semaphore_signal(
                barrier_sem, inc=1,
                device_id=(nbr,), device_id_type=pl.DeviceIdType.MESH,
            )
        pl.semaphore_wait(barrier_sem, 2)

        x_top = x_ref.at[pl.ds(0, m_half), :]
        x_bot = x_ref.at[pl.ds(m_half, m_half), :]
        my_top = out_ref.at[pl.ds(my_pos * m_per, m_half), :]
        my_bot = out_ref.at[pl.ds(my_pos * m_per + m_half, m_half), :]

        def remote(src, dst, s, r, tgt):
            return pltpu.make_async_remote_copy(
                src_ref=src, dst_ref=dst,
                send_sem=send_sems.at[s], recv_sem=recv_sems.at[r],
                device_id=(tgt,), device_id_type=pl.DeviceIdType.MESH,
            )

        a1 = remote(x_top, my_top, 0, 0, right)
        b1 = remote(x_bot, my_bot, 3, 3, left)
        a2 = remote(x_bot, my_bot, 1, 1, right)
        b2 = remote(x_top, my_top, 4, 4, left)
        a1.start()
        b1.start()
        a2.start()
        b2.start()

        local_copy = pltpu.make_async_copy(
            x_ref, out_ref.at[pl.ds(my_pos * m_per, m_per), :], local_sem
        )
        local_copy.start()

        a1.wait_recv()
        diag_top = out_ref.at[pl.ds(left * m_per, m_half), :]
        a3 = remote(diag_top, diag_top, 2, 2, right)
        a3.start()

        b1.wait_recv()
        diag_bot = out_ref.at[pl.ds(right * m_per + m_half, m_half), :]
        b3 = remote(diag_bot, diag_bot, 5, 5, left)
        b3.start()

        local_copy.wait()
        a1.wait_send()
        b1.wait_send()
        a2.wait()
        b2.wait()
        a3.wait()
        b3.wait()

    return pl.pallas_call(
        body,
        out_shape=jax.ShapeDtypeStruct((N_DEV * m_per, n), x.dtype),
        in_specs=[pl.BlockSpec(memory_space=pltpu.VMEM)],
        out_specs=pl.BlockSpec(memory_space=pltpu.MemorySpace.HBM),
        scratch_shapes=[
            pltpu.SemaphoreType.DMA((6,)),
            pltpu.SemaphoreType.DMA((6,)),
            pltpu.SemaphoreType.DMA,
        ],
        compiler_params=pltpu.CompilerParams(
            collective_id=0,
            skip_device_barrier=True,
            disable_bounds_checks=True,
        ),
    )(x)
